# baseline (speedup 1.0000x reference)
import functools
import numpy as np
import jax
import jax.numpy as jnp
from jax import lax
from jax.experimental import pallas as pl
from jax.experimental.pallas import tpu as pltpu
from jax.experimental.pallas import tpu_sc as plsc

_OUT_H = 7
_OUT_W = 7
_SCALE = 0.125
_G = 2
_O = 8
_NBIN = _OUT_H * _OUT_W
_NCHUNK = 7
_COLS = _NCHUNK * 128
_PCOL = 880
_RBLK = 64


def _const_table():
    cols = np.arange(_COLS)
    bin_ = cols >> 4
    s = (cols >> 2) & 3
    k = cols & 3
    h = np.minimum(bin_ // _OUT_W, _OUT_H - 1)
    w = bin_ % _OUT_W
    sh = s >> 1
    sw = s & 1
    t = np.zeros((8, _COLS), np.float32)
    t[0] = h
    t[1] = w
    t[2] = (sh + 0.5) / _G
    t[3] = (sw + 0.5) / _G
    t[4] = (k < 2)
    t[5] = (k % 2 == 0)
    t[6] = (bin_ < _NBIN)
    return jnp.asarray(t)


def _prep_body(rois_ref, tab_ref, idx_ref, w_ref, *, H, W):
    r = rois_ref[...]
    b = r[:, 0:1]
    cx = r[:, 1:2] * _SCALE
    cy = r[:, 2:3] * _SCALE
    rw = jnp.maximum(r[:, 3:4] * _SCALE, 1.0)
    rh = jnp.maximum(r[:, 4:5] * _SCALE, 1.0)
    th = r[:, 5:6]
    cos_t = jnp.cos(th)
    sin_t = jnp.sin(th)
    binh = rh / _OUT_H
    binw = rw / _OUT_W
    bh = tab_ref[0:1, :]
    bw = tab_ref[1:2, :]
    sy = tab_ref[2:3, :]
    sx = tab_ref[3:4, :]
    ysel = tab_ref[4:5, :]
    xsel = tab_ref[5:6, :]
    wmask = tab_ref[6:7, :]
    yy = rh * (-0.5) + (bh + sy) * binh
    xx = rw * (-0.5) + (bw + sx) * binw
    y = yy * cos_t - xx * sin_t + cy
    x = yy * sin_t + xx * cos_t + cx
    Hf = float(H)
    Wf = float(W)
    valid = ((y >= -1.0) & (y <= Hf) & (x >= -1.0) & (x <= Wf)).astype(jnp.float32)
    yc = jnp.maximum(y, 0.0)
    yl0 = jnp.floor(yc)
    condy = yl0 >= Hf - 1.0
    y_low = jnp.where(condy, Hf - 1.0, yl0)
    y_high = jnp.where(condy, Hf - 1.0, jnp.minimum(yl0 + 1.0, Hf - 1.0))
    yc = jnp.where(condy, Hf - 1.0, yc)
    ly = yc - y_low
    hy = 1.0 - ly
    xc = jnp.maximum(x, 0.0)
    xl0 = jnp.floor(xc)
    condx = xl0 >= Wf - 1.0
    x_low = jnp.where(condx, Wf - 1.0, xl0)
    x_high = jnp.where(condx, Wf - 1.0, jnp.minimum(xl0 + 1.0, Wf - 1.0))
    xc = jnp.where(condx, Wf - 1.0, xc)
    lx = xc - x_low
    hx = 1.0 - lx
    y_s = jnp.where(ysel > 0.0, y_low, y_high)
    wy = jnp.where(ysel > 0.0, hy, ly)
    x_s = jnp.where(xsel > 0.0, x_low, x_high)
    wx = jnp.where(xsel > 0.0, hx, lx)
    wgt = wy * wx * valid * (0.25 * wmask)
    idxf = b * (Hf * Wf) + y_s * Wf + x_s
    indf = th * (_O / (2.0 * np.pi))
    indfl = jnp.floor(indf)
    l_var = indf - indfl
    r_var = 1.0 - l_var
    ind_i = indfl - 8.0 * jnp.floor(indfl * 0.125)
    colid = lax.broadcasted_iota(jnp.int32, wgt.shape, 1)
    w_out = jnp.where(colid == _PCOL, r_var,
                      jnp.where(colid == _PCOL + 1, l_var, wgt))
    idx_out = jnp.where(colid < _NBIN * 16, idxf,
                        jnp.where(colid == _PCOL, ind_i, 0.0))
    idx_ref[...] = idx_out.astype(jnp.int32)
    w_ref[...] = w_out


def _sc_body(feats_hbm, idx_hbm, w_hbm, out_hbm,
             idx_v, w_v, rows_v, pooled_v, out_v, semA, semB,
             *, rois_per_tile):
    cid = lax.axis_index("c")
    sid = lax.axis_index("s")
    wid = sid * 2 + cid

    def start(c, buf, sem):
        pltpu.make_async_copy(feats_hbm.at[idx_v.at[c]], rows_v.at[buf], sem).start()

    def wait(buf, sem):
        pltpu.make_async_copy(feats_hbm.at[idx_v.at[0]], rows_v.at[buf], sem).wait()

    def compute(c, buf):
        def lb_body(lb, _):
            base = lb * 16
            wvec = w_v[c, pl.ds(base, 16)]
            ws = [wvec[k] for k in range(16)]
            binrow = (c * 8 + lb) * 256
            for j2 in range(8):
                sl = pl.ds(16 * j2, 16)
                accA = None
                accB = None
                for k in range(16):
                    wv = rows_v[buf, base + k, sl]
                    a = plsc.bitcast(lax.shift_left(wv, 16), jnp.float32)
                    bo = plsc.bitcast(wv & jnp.int32(-65536), jnp.float32)
                    if accA is None:
                        accA = ws[k] * a
                        accB = ws[k] * bo
                    else:
                        accA = accA + ws[k] * a
                        accB = accB + ws[k] * bo
                pooled_v[pl.ds(binrow + 32 * j2, 16)] = accA
                pooled_v[pl.ds(binrow + 32 * j2 + 16, 16)] = accB
            return 0
        lax.fori_loop(0, 8, lb_body, 0)

    def roi_body(i, _):
        roi = wid * rois_per_tile + i
        pltpu.sync_copy(idx_hbm.at[roi], idx_v)
        pltpu.sync_copy(w_hbm.at[roi], w_v)
        pvec_i = idx_v[6, pl.ds(112, 16)]
        pvec_w = w_v[6, pl.ds(112, 16)]
        ind = pvec_i[0]
        rv = pvec_w[0]
        lv = pvec_w[1]
        start(0, 0, semA)

        def pair_body(t, _):
            c0 = 2 * t
            start(c0 + 1, 1, semB)
            wait(0, semA)
            compute(c0, 0)
            start(c0 + 2, 0, semA)
            wait(1, semB)
            compute(c0 + 1, 1)
            return 0
        lax.fori_loop(0, 3, pair_body, 0)
        wait(0, semA)
        compute(6, 0)

        iota = lax.iota(jnp.int32, 16)
        for j in range(16):
            cvec = iota + 16 * j
            grp = cvec & (-8)
            o = cvec & 7
            sA = grp | ((o - ind) & 7)
            sB = grp | ((o - ind + 1) & 7)
            pA = (sA & (-32)) | ((sA & 1) << 4) | ((sA & 31) >> 1)
            pB = (sB & (-32)) | ((sB & 1) << 4) | ((sB & 31) >> 1)
            dstb = cvec * _NBIN

            def blend_body(bn, _):
                a = plsc.load_gather(pooled_v, [pA + bn * 256])
                bb = plsc.load_gather(pooled_v, [pB + bn * 256])
                plsc.store_scatter(out_v, [dstb + bn], rv * a + lv * bb)
                return 0
            lax.fori_loop(0, _NBIN, blend_body, 0)
        pltpu.sync_copy(out_v, out_hbm.at[roi])
        return 0
    lax.fori_loop(0, rois_per_tile, roi_body, 0)


def kernel(features, rois):
    N, C, H, W = features.shape
    R = rois.shape[0]
    feats = jnp.transpose(features, (0, 2, 3, 1)).reshape(N * H * W, C)
    feats = lax.bitcast_convert_type(
        feats.astype(jnp.bfloat16).reshape(N * H * W, C // 2, 2), jnp.int32)
    rois_p = jnp.pad(rois, ((0, 0), (0, 128 - rois.shape[1])))
    tab = _const_table()
    idx_all, w_all = pl.pallas_call(
        functools.partial(_prep_body, H=H, W=W),
        grid=(R // _RBLK,),
        in_specs=[
            pl.BlockSpec((_RBLK, 128), lambda i: (i, 0)),
            pl.BlockSpec((8, _COLS), lambda i: (0, 0)),
        ],
        out_specs=[
            pl.BlockSpec((_RBLK, _COLS), lambda i: (i, 0)),
            pl.BlockSpec((_RBLK, _COLS), lambda i: (i, 0)),
        ],
        out_shape=[
            jax.ShapeDtypeStruct((R, _COLS), jnp.int32),
            jax.ShapeDtypeStruct((R, _COLS), jnp.float32),
        ],
    )(rois_p, tab)
    idx3 = idx_all.reshape(R, _NCHUNK, 128)
    w3 = w_all.reshape(R, _NCHUNK, 128)

    rois_per_tile = R // 32
    mesh = plsc.VectorSubcoreMesh(core_axis_name="c", subcore_axis_name="s")
    out = pl.kernel(
        functools.partial(_sc_body, rois_per_tile=rois_per_tile),
        out_type=jax.ShapeDtypeStruct((R, C * _NBIN), jnp.float32),
        mesh=mesh,
        compiler_params=pltpu.CompilerParams(needs_layout_passes=False),
        scratch_types=[
            pltpu.VMEM((_NCHUNK, 128), jnp.int32),
            pltpu.VMEM((_NCHUNK, 128), jnp.float32),
            pltpu.VMEM((2, 128, 128), jnp.int32),
            pltpu.VMEM((_NCHUNK * 8 * 256,), jnp.float32),
            pltpu.VMEM((C * _NBIN,), jnp.float32),
            pltpu.SemaphoreType.DMA,
            pltpu.SemaphoreType.DMA,
        ],
    )(feats, idx3, w3)
    return out.reshape(R, C, _OUT_H, _OUT_W)


# speedup vs baseline: 1.8098x; 1.8098x over previous
import functools
import numpy as np
import jax
import jax.numpy as jnp
from jax import lax
from jax.experimental import pallas as pl
from jax.experimental.pallas import tpu as pltpu
from jax.experimental.pallas import tpu_sc as plsc

_OUT_H = 7
_OUT_W = 7
_SCALE = 0.125
_G = 2
_O = 8
_NBIN = _OUT_H * _OUT_W
_NCHUNK = 7
_COLS = _NCHUNK * 128
_PCOL = 880
_RBLK = 64


def _const_table():
    cols = np.arange(_COLS)
    bin_ = cols >> 4
    s = (cols >> 2) & 3
    k = cols & 3
    h = np.minimum(bin_ // _OUT_W, _OUT_H - 1)
    w = bin_ % _OUT_W
    sh = s >> 1
    sw = s & 1
    t = np.zeros((8, _COLS), np.float32)
    t[0] = h
    t[1] = w
    t[2] = (sh + 0.5) / _G
    t[3] = (sw + 0.5) / _G
    t[4] = (k < 2)
    t[5] = (k % 2 == 0)
    t[6] = (bin_ < _NBIN)
    return jnp.asarray(t)


def _prep_body(rois_ref, tab_ref, idx_ref, w_ref, *, H, W):
    r = rois_ref[...]
    b = r[:, 0:1]
    cx = r[:, 1:2] * _SCALE
    cy = r[:, 2:3] * _SCALE
    rw = jnp.maximum(r[:, 3:4] * _SCALE, 1.0)
    rh = jnp.maximum(r[:, 4:5] * _SCALE, 1.0)
    th = r[:, 5:6]
    cos_t = jnp.cos(th)
    sin_t = jnp.sin(th)
    binh = rh / _OUT_H
    binw = rw / _OUT_W
    bh = tab_ref[0:1, :]
    bw = tab_ref[1:2, :]
    sy = tab_ref[2:3, :]
    sx = tab_ref[3:4, :]
    ysel = tab_ref[4:5, :]
    xsel = tab_ref[5:6, :]
    wmask = tab_ref[6:7, :]
    yy = rh * (-0.5) + (bh + sy) * binh
    xx = rw * (-0.5) + (bw + sx) * binw
    y = yy * cos_t - xx * sin_t + cy
    x = yy * sin_t + xx * cos_t + cx
    Hf = float(H)
    Wf = float(W)
    valid = ((y >= -1.0) & (y <= Hf) & (x >= -1.0) & (x <= Wf)).astype(jnp.float32)
    yc = jnp.maximum(y, 0.0)
    yl0 = jnp.floor(yc)
    condy = yl0 >= Hf - 1.0
    y_low = jnp.where(condy, Hf - 1.0, yl0)
    y_high = jnp.where(condy, Hf - 1.0, jnp.minimum(yl0 + 1.0, Hf - 1.0))
    yc = jnp.where(condy, Hf - 1.0, yc)
    ly = yc - y_low
    hy = 1.0 - ly
    xc = jnp.maximum(x, 0.0)
    xl0 = jnp.floor(xc)
    condx = xl0 >= Wf - 1.0
    x_low = jnp.where(condx, Wf - 1.0, xl0)
    x_high = jnp.where(condx, Wf - 1.0, jnp.minimum(xl0 + 1.0, Wf - 1.0))
    xc = jnp.where(condx, Wf - 1.0, xc)
    lx = xc - x_low
    hx = 1.0 - lx
    wy = jnp.where(ysel > 0.0, hy, ly)
    wx = jnp.where(xsel > 0.0, hx, lx)
    wgt = wy * wx * valid * (0.25 * wmask)
    idxf = b * (Hf * Wf) + y_low * Wf + x_low
    indf = th * (_O / (2.0 * np.pi))
    indfl = jnp.floor(indf)
    l_var = indf - indfl
    r_var = 1.0 - l_var
    ind_i = indfl - 8.0 * jnp.floor(indfl * 0.125)
    colid = lax.broadcasted_iota(jnp.int32, wgt.shape, 1)
    w_out = jnp.where(colid == _PCOL, r_var,
                      jnp.where(colid == _PCOL + 1, l_var, wgt))
    idx_out = jnp.where(colid < _NBIN * 16, idxf,
                        jnp.where(colid == _PCOL, ind_i, 0.0))
    idx_ref[...] = idx_out.astype(jnp.int32)
    w_ref[...] = w_out


def _sc_body(feats_hbm, idx_hbm, w_hbm, out_hbm,
             idx_v, w_v, rows_v, pooled_v, out_v, semA, semB,
             *, rois_per_tile):
    cid = lax.axis_index("c")
    sid = lax.axis_index("s")
    wid = sid * 2 + cid

    def start(c, buf, sem):
        pltpu.make_async_copy(feats_hbm.at[idx_v.at[c]], rows_v.at[buf], sem).start()

    def wait(buf, sem):
        pltpu.make_async_copy(feats_hbm.at[idx_v.at[0]], rows_v.at[buf], sem).wait()

    def compute(c, buf):
        def lb_body(lb, _):
            base = lb * 16
            wvec = w_v[c, pl.ds(base, 16)]
            ws = [wvec[k] for k in range(16)]
            binrow = (c * 8 + lb) * 256
            for j2 in range(8):
                accA = None
                accB = None
                for s in range(4):
                    for seg in range(4):
                        wv = rows_v[buf, lb * 4 + s, pl.ds(seg * 128 + 16 * j2, 16)]
                        a = plsc.bitcast(lax.shift_left(wv, 16), jnp.float32)
                        bo = plsc.bitcast(wv & jnp.int32(-65536), jnp.float32)
                        wk = ws[s * 4 + seg]
                        if accA is None:
                            accA = wk * a
                            accB = wk * bo
                        else:
                            accA = accA + wk * a
                            accB = accB + wk * bo
                pooled_v[pl.ds(binrow + 32 * j2, 16)] = accA
                pooled_v[pl.ds(binrow + 32 * j2 + 16, 16)] = accB
            return 0
        lax.fori_loop(0, 8, lb_body, 0)

    def roi_body(i, _):
        roi = wid * rois_per_tile + i
        pltpu.sync_copy(idx_hbm.at[roi], idx_v)
        pltpu.sync_copy(w_hbm.at[roi], w_v)
        pvec_i = idx_v[6, pl.ds(16, 16)]
        pvec_w = w_v[6, pl.ds(112, 16)]
        ind = pvec_i[12]
        rv = pvec_w[0]
        lv = pvec_w[1]
        start(0, 0, semA)

        def pair_body(t, _):
            c0 = 2 * t
            start(c0 + 1, 1, semB)
            wait(0, semA)
            compute(c0, 0)
            start(c0 + 2, 0, semA)
            wait(1, semB)
            compute(c0 + 1, 1)
            return 0
        lax.fori_loop(0, 3, pair_body, 0)
        wait(0, semA)
        compute(6, 0)

        iota = lax.iota(jnp.int32, 16)
        for j in range(16):
            cvec = iota + 16 * j
            grp = cvec & (-8)
            o = cvec & 7
            sA = grp | ((o - ind) & 7)
            sB = grp | ((o - ind + 1) & 7)
            pA = (sA & (-32)) | ((sA & 1) << 4) | ((sA & 31) >> 1)
            pB = (sB & (-32)) | ((sB & 1) << 4) | ((sB & 31) >> 1)
            dstb = cvec * _NBIN

            def blend_body(bn, _):
                a = plsc.load_gather(pooled_v, [pA + bn * 256])
                bb = plsc.load_gather(pooled_v, [pB + bn * 256])
                plsc.store_scatter(out_v, [dstb + bn], rv * a + lv * bb)
                return 0
            lax.fori_loop(0, _NBIN, blend_body, 0)
        pltpu.sync_copy(out_v, out_hbm.at[roi])
        return 0
    lax.fori_loop(0, rois_per_tile, roi_body, 0)


def kernel(features, rois):
    N, C, H, W = features.shape
    R = rois.shape[0]
    NHW = N * H * W
    fb = jnp.transpose(features, (0, 2, 3, 1)).reshape(NHW, C).astype(jnp.bfloat16)
    fb = jnp.pad(fb, ((0, W + 2), (0, 0)))
    quad = jnp.concatenate(
        [fb[0:NHW], fb[1:NHW + 1], fb[W:NHW + W], fb[W + 1:NHW + W + 1]], axis=1)
    feats = lax.bitcast_convert_type(quad.reshape(NHW, 2 * C, 2), jnp.int32)
    rois_p = jnp.pad(rois, ((0, 0), (0, 128 - rois.shape[1])))
    tab = _const_table()
    idx_all, w_all = pl.pallas_call(
        functools.partial(_prep_body, H=H, W=W),
        grid=(R // _RBLK,),
        in_specs=[
            pl.BlockSpec((_RBLK, 128), lambda i: (i, 0)),
            pl.BlockSpec((8, _COLS), lambda i: (0, 0)),
        ],
        out_specs=[
            pl.BlockSpec((_RBLK, _COLS), lambda i: (i, 0)),
            pl.BlockSpec((_RBLK, _COLS), lambda i: (i, 0)),
        ],
        out_shape=[
            jax.ShapeDtypeStruct((R, _COLS), jnp.int32),
            jax.ShapeDtypeStruct((R, _COLS), jnp.float32),
        ],
    )(rois_p, tab)
    idx3 = idx_all.reshape(R, _NCHUNK * 32, 4)[:, :, 0].reshape(R, _NCHUNK, 32)
    w3 = w_all.reshape(R, _NCHUNK, 128)

    rois_per_tile = R // 32
    mesh = plsc.VectorSubcoreMesh(core_axis_name="c", subcore_axis_name="s")
    out = pl.kernel(
        functools.partial(_sc_body, rois_per_tile=rois_per_tile),
        out_type=jax.ShapeDtypeStruct((R, C * _NBIN), jnp.float32),
        mesh=mesh,
        compiler_params=pltpu.CompilerParams(needs_layout_passes=False),
        scratch_types=[
            pltpu.VMEM((_NCHUNK, 32), jnp.int32),
            pltpu.VMEM((_NCHUNK, 128), jnp.float32),
            pltpu.VMEM((2, 32, 512), jnp.int32),
            pltpu.VMEM((_NCHUNK * 8 * 256,), jnp.float32),
            pltpu.VMEM((C * _NBIN,), jnp.float32),
            pltpu.SemaphoreType.DMA,
            pltpu.SemaphoreType.DMA,
        ],
    )(feats, idx3, w3)
    return out.reshape(R, C, _OUT_H, _OUT_W)
